# Initial kernel scaffold; baseline (speedup 1.0000x reference)
import jax
import jax.numpy as jnp
from jax.experimental import pallas as pl


def kernel(x, dw_w, dw_b, pw_w, pw_b):
    raise NotImplementedError("write your pallas kernel here")


# speedup vs baseline: 1.4218x; 1.4218x over previous
import functools

import jax
import jax.numpy as jnp
from jax.experimental import pallas as pl
from jax.experimental.pallas import tpu as pltpu

KS = 5
PAD = KS // 2


def _pair_kernel(x_ref, dww_ref, dwb_ref, wblk_ref, pwb_ref, out_ref, acc_ref,
                 *, H, W, HWp, R, n_chunks):
    f32 = jnp.float32
    bf16 = jnp.bfloat16
    CR = R // n_chunks

    lane = jax.lax.broadcasted_iota(jnp.int32, (1, HWp), 1)
    h_idx = lane // W
    w_idx = lane % W
    taps = []
    for ky in range(KS):
        dy = ky - PAD
        row_ok = jnp.logical_and(h_idx + dy >= 0, h_idx + dy < H)
        for kx in range(KS):
            dx = kx - PAD
            if dy == 0 and dx == 0:
                continue
            col_ok = jnp.logical_and(w_idx + dx >= 0, w_idx + dx < W)
            d = dy * W + dx
            taps.append((ky * KS + kx, (-d) % HWp,
                         jnp.logical_and(row_ok, col_ok)))

    t_center = (KS // 2) * KS + KS // 2
    for c in range(n_chunks):
        r0 = c * CR
        xc = x_ref[pl.ds(r0, CR), :]
        r16 = jnp.maximum(xc, 0.0).astype(bf16)
        packed = pltpu.bitcast(r16, jnp.int32)
        acc_a = r16 * dww_ref[t_center, pl.ds(r0, CR), :]
        acc_b = None
        for i, (t, shift, valid) in enumerate(taps):
            rolled = pltpu.roll(packed, shift, axis=1)
            masked = jnp.where(valid, rolled, 0)
            mb = pltpu.bitcast(masked, bf16)
            term = mb * dww_ref[t, pl.ds(r0, CR), :]
            if i % 2 == 0:
                acc_a = acc_a + term
            else:
                acc_b = term if acc_b is None else acc_b + term
        dsum = acc_a.astype(f32) + acc_b.astype(f32)
        full = xc + dwb_ref[pl.ds(r0, CR), :] + dsum
        acc_ref[pl.ds(r0, CR), :] = full.astype(bf16)

    out_ref[...] = (jnp.dot(wblk_ref[...], acc_ref[...],
                            preferred_element_type=f32)
                    + pwb_ref[...]).astype(out_ref.dtype)


@jax.jit
def _resblock2d_fast(x_nchw, dw_w, dw_b, pw_w, pw_b):
    N, Cin, H, W = x_nchw.shape
    Cout = pw_w.shape[1]
    HW = H * W
    HWp = ((HW + 127) // 128) * 128
    R = 2 * Cin
    Ro = 2 * Cout
    n_chunks = 4 if (R % 4 == 0 and (R // 4) % 2 == 0) else 1

    f32 = jnp.float32
    bf16 = jnp.bfloat16

    x2 = x_nchw.reshape(N * Cin, HW)
    if HWp != HW:
        x2 = jnp.pad(x2, ((0, 0), (0, HWp - HW)))

    dww2 = jnp.concatenate([dw_w, dw_w], axis=1).astype(bf16)[:, :, None]
    dwb2 = jnp.concatenate([dw_b, dw_b]).astype(f32)[:, None]
    wblk = jnp.kron(jnp.eye(2, dtype=f32), pw_w.T).astype(bf16)
    pwb2 = jnp.concatenate([pw_b, pw_b]).astype(f32)[:, None]

    body = functools.partial(_pair_kernel, H=H, W=W, HWp=HWp, R=R,
                             n_chunks=n_chunks)

    out2 = pl.pallas_call(
        body,
        out_shape=jax.ShapeDtypeStruct((N * Cout, HWp), x_nchw.dtype),
        grid=(N // 2,),
        in_specs=[
            pl.BlockSpec((R, HWp), lambda i: (i, 0)),
            pl.BlockSpec((KS * KS, R, 1), lambda i: (0, 0, 0)),
            pl.BlockSpec((R, 1), lambda i: (0, 0)),
            pl.BlockSpec((Ro, R), lambda i: (0, 0)),
            pl.BlockSpec((Ro, 1), lambda i: (0, 0)),
        ],
        out_specs=pl.BlockSpec((Ro, HWp), lambda i: (i, 0)),
        scratch_shapes=[pltpu.VMEM((R, HWp), bf16)],
        compiler_params=pltpu.CompilerParams(
            dimension_semantics=("parallel",),
            vmem_limit_bytes=48 * 1024 * 1024,
        ),
    )(x2, dww2, dwb2, wblk, pwb2)

    if HWp != HW:
        out2 = out2[:, :HW]
    return out2.reshape(N, Cout, H, W)


def kernel(x, dw_w, dw_b, pw_w, pw_b):
    return _resblock2d_fast(x, dw_w, dw_b, pw_w, pw_b)


# speedup vs baseline: 1.5878x; 1.1168x over previous
import functools

import jax
import jax.numpy as jnp
from jax.experimental import pallas as pl
from jax.experimental.pallas import tpu as pltpu

KS = 5
PAD = KS // 2


def _pair_kernel(x_ref, dww_ref, dwb_ref, wblk_ref, pwb_ref, out_ref, acc_ref,
                 *, H, W, HWp, R, n_chunks):
    f32 = jnp.float32
    bf16 = jnp.bfloat16
    CR = R // n_chunks
    Ro = out_ref.shape[0]

    lane = jax.lax.broadcasted_iota(jnp.int32, (1, HWp), 1)
    h_idx = lane // W
    w_idx = lane % W
    taps = []
    for ky in range(KS):
        dy = ky - PAD
        row_ok = jnp.logical_and(h_idx + dy >= 0, h_idx + dy < H)
        for kx in range(KS):
            dx = kx - PAD
            if dy == 0 and dx == 0:
                continue
            col_ok = jnp.logical_and(w_idx + dx >= 0, w_idx + dx < W)
            d = dy * W + dx
            taps.append((ky * KS + kx, (-d) % HWp,
                         jnp.logical_and(row_ok, col_ok)))

    t_center = (KS // 2) * KS + KS // 2
    for c in range(n_chunks):
        r0 = c * CR
        xc = x_ref[pl.ds(r0, CR), :, :].reshape(CR, H * W)
        r16 = jnp.maximum(xc, 0.0).astype(bf16)
        packed = pltpu.bitcast(r16, jnp.int32)
        acc_a = r16 * dww_ref[t_center, pl.ds(r0, CR), :]
        acc_b = None
        for i, (t, shift, valid) in enumerate(taps):
            rolled = pltpu.roll(packed, shift, axis=1)
            masked = jnp.where(valid, rolled, 0)
            mb = pltpu.bitcast(masked, bf16)
            term = mb * dww_ref[t, pl.ds(r0, CR), :]
            if i % 2 == 0:
                acc_a = acc_a + term
            else:
                acc_b = term if acc_b is None else acc_b + term
        dsum = acc_a.astype(f32) + acc_b.astype(f32)
        full = xc + dwb_ref[pl.ds(r0, CR), :] + dsum
        acc_ref[pl.ds(r0, CR), :] = full.astype(bf16)

    res = (jnp.dot(wblk_ref[...], acc_ref[...], preferred_element_type=f32)
           + pwb_ref[...]).astype(out_ref.dtype)
    out_ref[...] = res.reshape(Ro, H, W)


@jax.jit
def _resblock2d_fast(x_nchw, dw_w, dw_b, pw_w, pw_b):
    N, Cin, H, W = x_nchw.shape
    Cout = pw_w.shape[1]
    HW = H * W
    HWp = HW
    R = 2 * Cin
    Ro = 2 * Cout
    n_chunks = 4 if (R % 4 == 0 and (R // 4) % 2 == 0) else 1

    f32 = jnp.float32
    bf16 = jnp.bfloat16

    x3 = x_nchw.reshape(N * Cin, H, W)

    dww2 = jnp.concatenate([dw_w, dw_w], axis=1).astype(bf16)[:, :, None]
    dwb2 = jnp.concatenate([dw_b, dw_b]).astype(f32)[:, None]
    wblk = jnp.kron(jnp.eye(2, dtype=f32), pw_w.T).astype(bf16)
    pwb2 = jnp.concatenate([pw_b, pw_b]).astype(f32)[:, None]

    body = functools.partial(_pair_kernel, H=H, W=W, HWp=HWp, R=R,
                             n_chunks=n_chunks)

    out3 = pl.pallas_call(
        body,
        out_shape=jax.ShapeDtypeStruct((N * Cout, H, W), x_nchw.dtype),
        grid=(N // 2,),
        in_specs=[
            pl.BlockSpec((R, H, W), lambda i: (i, 0, 0)),
            pl.BlockSpec((KS * KS, R, 1), lambda i: (0, 0, 0)),
            pl.BlockSpec((R, 1), lambda i: (0, 0)),
            pl.BlockSpec((Ro, R), lambda i: (0, 0)),
            pl.BlockSpec((Ro, 1), lambda i: (0, 0)),
        ],
        out_specs=pl.BlockSpec((Ro, H, W), lambda i: (i, 0, 0)),
        scratch_shapes=[pltpu.VMEM((R, HWp), bf16)],
        compiler_params=pltpu.CompilerParams(
            dimension_semantics=("parallel",),
            vmem_limit_bytes=48 * 1024 * 1024,
        ),
    )(x3, dww2, dwb2, wblk, pwb2)

    return out3.reshape(N, Cout, H, W)


def kernel(x, dw_w, dw_b, pw_w, pw_b):
    return _resblock2d_fast(x, dw_w, dw_b, pw_w, pw_b)


# speedup vs baseline: 1.8527x; 1.1668x over previous
import functools

import jax
import jax.numpy as jnp
from jax.experimental import pallas as pl
from jax.experimental.pallas import tpu as pltpu

KS = 5
PAD = KS // 2


def _pair_kernel(x_ref, dww_ref, dwb_ref, wblk_ref, pwb_ref, out_ref, acc_ref,
                 *, H, W, HWp, R, n_chunks):
    f32 = jnp.float32
    bf16 = jnp.bfloat16
    CR = R // n_chunks
    Ro = out_ref.shape[0]

    lane = jax.lax.broadcasted_iota(jnp.int32, (1, HWp), 1)
    h_idx = lane // W
    w_idx = lane % W
    taps = []
    for ky in range(KS):
        dy = ky - PAD
        row_ok = jnp.logical_and(h_idx + dy >= 0, h_idx + dy < H)
        for kx in range(KS):
            dx = kx - PAD
            if dy == 0 and dx == 0:
                continue
            col_ok = jnp.logical_and(w_idx + dx >= 0, w_idx + dx < W)
            d = dy * W + dx
            taps.append((ky * KS + kx, (-d) % HWp,
                         jnp.logical_and(row_ok, col_ok)))

    t_center = (KS // 2) * KS + KS // 2
    for c in range(n_chunks):
        r0 = c * CR
        xb = x_ref[pl.ds(r0, CR), :, :].astype(bf16).reshape(CR, H * W)
        r16 = jnp.maximum(xb, 0)
        packed = pltpu.bitcast(r16, jnp.int32)
        if HWp % 128 == 0:
            nrep = HWp // 128
            wide = lambda t: pltpu.repeat(
                dww_ref[t, pl.ds(r0, CR), :], nrep, axis=1)
        else:
            wide = lambda t: dww_ref[t, pl.ds(r0, CR), 0:1]
        acc_a = r16 * wide(t_center)
        acc_b = None
        for i, (t, shift, valid) in enumerate(taps):
            rolled = pltpu.roll(packed, shift, axis=1)
            masked = jnp.where(valid, rolled, 0)
            mb = pltpu.bitcast(masked, bf16)
            term = mb * wide(t)
            if i % 2 == 0:
                acc_a = acc_a + term
            else:
                acc_b = term if acc_b is None else acc_b + term
        full = (xb + dwb_ref[pl.ds(r0, CR), :]) + (acc_a + acc_b)
        acc_ref[pl.ds(r0, CR), :] = full

    res = (jnp.dot(wblk_ref[...], acc_ref[...], preferred_element_type=f32)
           + pwb_ref[...]).astype(out_ref.dtype)
    out_ref[...] = res.reshape(Ro, H, W)


@jax.jit
def _resblock2d_fast(x_nchw, dw_w, dw_b, pw_w, pw_b):
    N, Cin, H, W = x_nchw.shape
    Cout = pw_w.shape[1]
    HW = H * W
    HWp = HW
    R = 2 * Cin
    Ro = 2 * Cout
    n_chunks = 4 if (R % 4 == 0 and (R // 4) % 2 == 0) else 1

    f32 = jnp.float32
    bf16 = jnp.bfloat16

    x3 = x_nchw.reshape(N * Cin, H, W)

    dww2 = jnp.broadcast_to(
        jnp.concatenate([dw_w, dw_w], axis=1).astype(bf16)[:, :, None],
        (KS * KS, R, 128))
    dwb2 = jnp.concatenate([dw_b, dw_b]).astype(bf16)[:, None]
    wblk = jnp.kron(jnp.eye(2, dtype=f32), pw_w.T).astype(bf16)
    pwb2 = jnp.concatenate([pw_b, pw_b]).astype(f32)[:, None]

    body = functools.partial(_pair_kernel, H=H, W=W, HWp=HWp, R=R,
                             n_chunks=n_chunks)

    out3 = pl.pallas_call(
        body,
        out_shape=jax.ShapeDtypeStruct((N * Cout, H, W), x_nchw.dtype),
        grid=(N // 2,),
        in_specs=[
            pl.BlockSpec((R, H, W), lambda i: (i, 0, 0)),
            pl.BlockSpec((KS * KS, R, 128), lambda i: (0, 0, 0)),
            pl.BlockSpec((R, 1), lambda i: (0, 0)),
            pl.BlockSpec((Ro, R), lambda i: (0, 0)),
            pl.BlockSpec((Ro, 1), lambda i: (0, 0)),
        ],
        out_specs=pl.BlockSpec((Ro, H, W), lambda i: (i, 0, 0)),
        scratch_shapes=[pltpu.VMEM((R, HWp), bf16)],
        compiler_params=pltpu.CompilerParams(
            dimension_semantics=("parallel",),
            vmem_limit_bytes=48 * 1024 * 1024,
        ),
    )(x3, dww2, dwb2, wblk, pwb2)

    return out3.reshape(N, Cout, H, W)


def kernel(x, dw_w, dw_b, pw_w, pw_b):
    return _resblock2d_fast(x, dw_w, dw_b, pw_w, pw_b)
